# baseline (speedup 1.0000x reference)
import functools

import jax
import jax.numpy as jnp
from jax import lax
from jax.experimental import pallas as pl
from jax.experimental.pallas import tpu as pltpu
from jax.experimental.pallas import tpu_sc as plsc

N = 10000
E = 320000
F_IN = 128
H = 64
C = 40

NC = 2
NS = 16
NW = NC * NS
CK = 125
EW = E // NW
NCHUNK = EW // CK
N_PAD = 10240
RPT = N_PAD // NS
DEGW = 16

BR = 512


def _lin1_body(x_ref, wl_ref, wr_ref, b_ref, o_ref):
    x = x_ref[...]
    yl = jnp.dot(x, wl_ref[...], preferred_element_type=jnp.float32)
    yr = jnp.dot(x, wr_ref[...], preferred_element_type=jnp.float32)
    o_ref[...] = jnp.concatenate([yl, yr + b_ref[...]], axis=1)


def _lin1(x, wl, wr, b):
    return pl.pallas_call(
        _lin1_body,
        grid=(pl.cdiv(N, BR),),
        in_specs=[
            pl.BlockSpec((BR, F_IN), lambda i: (i, 0)),
            pl.BlockSpec((F_IN, H), lambda i: (0, 0)),
            pl.BlockSpec((F_IN, H), lambda i: (0, 0)),
            pl.BlockSpec((1, H), lambda i: (0, 0)),
        ],
        out_specs=pl.BlockSpec((BR, 2 * H), lambda i: (i, 0)),
        out_shape=jax.ShapeDtypeStruct((N, 2 * H), jnp.float32),
    )(x, wl, wr, b)


def _mid_body(a_ref, d_ref, p_ref, wl_ref, wr_ref, b_ref, o_ref):
    deg = jnp.maximum(d_ref[0, :, :1] + d_ref[1, :, :1], 1.0)
    h = jnp.maximum(
        (a_ref[0, :, :H] + a_ref[1, :, :H]) / deg + p_ref[:, H:], 0.0)
    yl = jnp.dot(h, wl_ref[...], preferred_element_type=jnp.float32)
    yr = jnp.dot(h, wr_ref[...], preferred_element_type=jnp.float32)
    o_ref[...] = jnp.concatenate([yl, yr + b_ref[...]], axis=1)


def _mid(acc, deg, packed, wl, wr, b):
    return pl.pallas_call(
        _mid_body,
        grid=(pl.cdiv(N, BR),),
        in_specs=[
            pl.BlockSpec((NC, BR, 2 * H), lambda i: (0, i, 0)),
            pl.BlockSpec((NC, BR, 8 * DEGW), lambda i: (0, i, 0)),
            pl.BlockSpec((BR, 2 * H), lambda i: (i, 0)),
            pl.BlockSpec((H, H), lambda i: (0, 0)),
            pl.BlockSpec((H, H), lambda i: (0, 0)),
            pl.BlockSpec((1, H), lambda i: (0, 0)),
        ],
        out_specs=pl.BlockSpec((BR, 2 * H), lambda i: (i, 0)),
        out_shape=jax.ShapeDtypeStruct((N, 2 * H), jnp.float32),
    )(acc, deg, packed, wl, wr, b)


def _fin_body(a_ref, d_ref, p_ref, w_ref, b_ref, o_ref):
    deg = jnp.maximum(d_ref[0, :, :1] + d_ref[1, :, :1], 1.0)
    h = jnp.maximum(
        (a_ref[0, :, :H] + a_ref[1, :, :H]) / deg + p_ref[:, H:], 0.0)
    o_ref[...] = jnp.dot(h, w_ref[...],
                         preferred_element_type=jnp.float32) + b_ref[...]


def _fin(acc, deg, packed, w, b):
    return pl.pallas_call(
        _fin_body,
        grid=(pl.cdiv(N, BR),),
        in_specs=[
            pl.BlockSpec((NC, BR, 2 * H), lambda i: (0, i, 0)),
            pl.BlockSpec((NC, BR, 8 * DEGW), lambda i: (0, i, 0)),
            pl.BlockSpec((BR, 2 * H), lambda i: (i, 0)),
            pl.BlockSpec((H, C), lambda i: (0, 0)),
            pl.BlockSpec((1, C), lambda i: (0, 0)),
        ],
        out_specs=pl.BlockSpec((BR, C), lambda i: (i, 0)),
        out_shape=jax.ShapeDtypeStruct((N, C), jnp.float32),
    )(acc, deg, packed, w, b)


NBUF = 2
GLEAD = 1
SLAG = NBUF - GLEAD
NOUTER = NCHUNK // NBUF
RPT_T = N // NS


def _pipeline(table, src_v, dst_v, rows_v, acc_s, gsem, ssem,
              ones_v=None, deg_s=None, dsem=None):
    with_deg = deg_s is not None

    def gstart(j, b):
        pltpu.async_copy(table.at[src_v.at[j]], rows_v.at[b], gsem.at[b])

    def step(j, b):
        if with_deg:
            pltpu.async_copy(ones_v, deg_s.at[dst_v.at[j]], dsem, add=True)
        pltpu.make_async_copy(
            table.at[src_v.at[j]], rows_v.at[b], gsem.at[b]).wait()
        pltpu.async_copy(
            rows_v.at[b], acc_s.at[dst_v.at[j]], ssem.at[b], add=True)

    def swait(j, b):
        pltpu.make_async_copy(
            rows_v.at[b], acc_s.at[dst_v.at[j]], ssem.at[b]).wait()

    def dwait(j):
        pltpu.make_async_copy(ones_v, deg_s.at[dst_v.at[j]], dsem).wait()

    for j in range(GLEAD):
        gstart(j, j % NBUF)
    for j in range(NBUF):
        if j >= SLAG:
            swait(j - SLAG, (j - SLAG) % NBUF)
        gstart(j + GLEAD, (j + GLEAD) % NBUF)
        step(j, j % NBUF)
        if with_deg and j >= 1:
            dwait(j - 1)

    def outer(i, carry):
        base = i * NBUF
        for b in range(NBUF):
            j = base + b
            swait(j - SLAG, (b - SLAG) % NBUF)
            gstart(j + GLEAD, (b + GLEAD) % NBUF)
            step(j, b)
            if with_deg:
                dwait(j - 1)
        return carry

    lax.fori_loop(1, NOUTER - 1, outer, 0)

    base = NCHUNK - NBUF
    for b in range(NBUF):
        j = base + b
        swait(j - SLAG, (j - SLAG) % NBUF)
        if j + GLEAD < NCHUNK:
            gstart(j + GLEAD, (j + GLEAD) % NBUF)
        step(j, b)
        if with_deg:
            dwait(j - 1)
    for j in range(NCHUNK - SLAG, NCHUNK):
        swait(j, j % NBUF)
    if with_deg:
        dwait(NCHUNK - 1)


def _agg_deg_body(table, ei_h, zrow_h, zdeg_h, acc_out, deg_out,
                  src_v, dst_v, rows_v, ones_v, table_s, acc_s, deg_s,
                  gsem, ssem, dsem):
    cid = lax.axis_index("c")
    sid = lax.axis_index("s")
    wid = sid * NC + cid
    pltpu.sync_copy(ei_h.at[0, wid], src_v)
    pltpu.sync_copy(ei_h.at[1, wid], dst_v)
    t0 = sid * RPT_T
    pltpu.sync_copy(table.at[pl.ds(t0, RPT_T), 0], table_s.at[pl.ds(t0, RPT_T)])
    r0 = sid * RPT
    pltpu.sync_copy(zrow_h, acc_s.at[pl.ds(r0, RPT)])
    pltpu.sync_copy(zdeg_h, deg_s.at[pl.ds(r0, RPT)])
    one = jnp.ones((DEGW,), jnp.float32)
    for r in range(CK):
        ones_v[r, :] = one
    plsc.subcore_barrier()
    _pipeline(table_s, src_v, dst_v, rows_v, acc_s, gsem, ssem,
              ones_v, deg_s, dsem)
    plsc.subcore_barrier()
    pltpu.sync_copy(acc_s.at[pl.ds(r0, RPT)],
                    acc_out.at[cid, pl.ds(r0, RPT), 0])
    pltpu.sync_copy(deg_s.at[pl.ds(r0, RPT)],
                    deg_out.at[cid, pl.ds(r0, RPT), 0])


def _agg_body(table, ei_h, zrow_h, acc_out,
              src_v, dst_v, rows_v, table_s, acc_s, gsem, ssem):
    cid = lax.axis_index("c")
    sid = lax.axis_index("s")
    wid = sid * NC + cid
    pltpu.sync_copy(ei_h.at[0, wid], src_v)
    pltpu.sync_copy(ei_h.at[1, wid], dst_v)
    t0 = sid * RPT_T
    pltpu.sync_copy(table.at[pl.ds(t0, RPT_T), 0], table_s.at[pl.ds(t0, RPT_T)])
    r0 = sid * RPT
    pltpu.sync_copy(zrow_h, acc_s.at[pl.ds(r0, RPT)])
    plsc.subcore_barrier()
    _pipeline(table_s, src_v, dst_v, rows_v, acc_s, gsem, ssem)
    plsc.subcore_barrier()
    pltpu.sync_copy(acc_s.at[pl.ds(r0, RPT)],
                    acc_out.at[cid, pl.ds(r0, RPT), 0])


_SC_MESH = dict(core_axis_name="c", subcore_axis_name="s")


def _agg_deg(table, ei, zrow, zdeg):
    return pl.kernel(
        _agg_deg_body,
        out_type=(
            jax.ShapeDtypeStruct((NC, N_PAD, 2, H), jnp.float32),
            jax.ShapeDtypeStruct((NC, N_PAD, 8, DEGW), jnp.float32),
        ),
        mesh=plsc.VectorSubcoreMesh(**_SC_MESH),
        compiler_params=pltpu.CompilerParams(use_tc_tiling_on_sc=False),
        scratch_types=[
            pltpu.VMEM((NCHUNK, CK), jnp.int32),
            pltpu.VMEM((NCHUNK, CK), jnp.int32),
            pltpu.VMEM((NBUF, CK, H), jnp.float32),
            pltpu.VMEM((CK, DEGW), jnp.float32),
            pltpu.VMEM_SHARED((N, H), jnp.float32),
            pltpu.VMEM_SHARED((N_PAD, H), jnp.float32),
            pltpu.VMEM_SHARED((N_PAD, DEGW), jnp.float32),
            pltpu.SemaphoreType.DMA((NBUF,)),
            pltpu.SemaphoreType.DMA((NBUF,)),
            pltpu.SemaphoreType.DMA,
        ],
    )(table, ei, zrow, zdeg)


def _agg(table, ei, zrow):
    return pl.kernel(
        _agg_body,
        out_type=jax.ShapeDtypeStruct((NC, N_PAD, 2, H), jnp.float32),
        mesh=plsc.VectorSubcoreMesh(**_SC_MESH),
        compiler_params=pltpu.CompilerParams(use_tc_tiling_on_sc=False),
        scratch_types=[
            pltpu.VMEM((NCHUNK, CK), jnp.int32),
            pltpu.VMEM((NCHUNK, CK), jnp.int32),
            pltpu.VMEM((NBUF, CK, H), jnp.float32),
            pltpu.VMEM_SHARED((N, H), jnp.float32),
            pltpu.VMEM_SHARED((N_PAD, H), jnp.float32),
            pltpu.SemaphoreType.DMA((NBUF,)),
            pltpu.SemaphoreType.DMA((NBUF,)),
        ],
    )(table, ei, zrow)


def kernel(x, edge_index, batch, W1l, b1, W1r, W2l, b2, W2r, Wlin, blin):
    ei = edge_index.reshape(2, NW, NCHUNK, CK)
    zrow = jnp.zeros((RPT, H), jnp.float32)
    zdeg = jnp.zeros((RPT, DEGW), jnp.float32)

    packed1 = _lin1(x, W1l, W1r, b1.reshape(1, H))
    acc1, deg1 = _agg_deg(packed1.reshape(N, 2, H), ei, zrow, zdeg)
    acc1 = acc1.reshape(NC, N_PAD, 2 * H)
    deg1 = deg1.reshape(NC, N_PAD, 8 * DEGW)
    packed2 = _mid(acc1, deg1, packed1, W2l, W2r, b2.reshape(1, H))
    acc2 = _agg(packed2.reshape(N, 2, H), ei, zrow)
    acc2 = acc2.reshape(NC, N_PAD, 2 * H)
    return _fin(acc2, deg1, packed2, Wlin, blin.reshape(1, C))


# speedup vs baseline: 1.8253x; 1.8253x over previous
import functools

import jax
import jax.numpy as jnp
from jax import lax
from jax.experimental import pallas as pl
from jax.experimental.pallas import tpu as pltpu
from jax.experimental.pallas import tpu_sc as plsc

N = 10000
E = 320000
F_IN = 128
H = 64
C = 40

NC = 2
NS = 16
NW = NC * NS
CK = 125
EW = E // NW
NCHUNK = EW // CK
N_PAD = 10240
RPT = N_PAD // NS
DEGW = 16

BR = 512


def _lin1_body(x_ref, wl_ref, wr_ref, b_ref, l_ref, r_ref):
    x = x_ref[...]
    l_ref[...] = jnp.dot(x, wl_ref[...], preferred_element_type=jnp.float32)
    r_ref[...] = jnp.dot(
        x, wr_ref[...], preferred_element_type=jnp.float32) + b_ref[...]


def _lin1(x, wl, wr, b):
    return pl.pallas_call(
        _lin1_body,
        grid=(pl.cdiv(N, BR),),
        in_specs=[
            pl.BlockSpec((BR, F_IN), lambda i: (i, 0)),
            pl.BlockSpec((F_IN, H), lambda i: (0, 0)),
            pl.BlockSpec((F_IN, H), lambda i: (0, 0)),
            pl.BlockSpec((1, H), lambda i: (0, 0)),
        ],
        out_specs=[
            pl.BlockSpec((BR, H), lambda i: (i, 0)),
            pl.BlockSpec((BR, H), lambda i: (i, 0)),
        ],
        out_shape=[
            jax.ShapeDtypeStruct((N, H), jnp.float32),
            jax.ShapeDtypeStruct((N, H), jnp.float32),
        ],
    )(x, wl, wr, b)


def _mid_body(a_ref, d_ref, xr_ref, wl_ref, wr_ref, b_ref, l_ref, r_ref):
    deg = jnp.maximum(d_ref[0, :, :1] + d_ref[1, :, :1], 1.0)
    h = jnp.maximum((a_ref[0] + a_ref[1]) / deg + xr_ref[...], 0.0)
    l_ref[...] = jnp.dot(h, wl_ref[...], preferred_element_type=jnp.float32)
    r_ref[...] = jnp.dot(
        h, wr_ref[...], preferred_element_type=jnp.float32) + b_ref[...]


def _mid(acc, deg, xr, wl, wr, b):
    return pl.pallas_call(
        _mid_body,
        grid=(pl.cdiv(N, BR),),
        in_specs=[
            pl.BlockSpec((NC, BR, H), lambda i: (0, i, 0)),
            pl.BlockSpec((NC, BR, DEGW), lambda i: (0, i, 0)),
            pl.BlockSpec((BR, H), lambda i: (i, 0)),
            pl.BlockSpec((H, H), lambda i: (0, 0)),
            pl.BlockSpec((H, H), lambda i: (0, 0)),
            pl.BlockSpec((1, H), lambda i: (0, 0)),
        ],
        out_specs=[
            pl.BlockSpec((BR, H), lambda i: (i, 0)),
            pl.BlockSpec((BR, H), lambda i: (i, 0)),
        ],
        out_shape=[
            jax.ShapeDtypeStruct((N, H), jnp.float32),
            jax.ShapeDtypeStruct((N, H), jnp.float32),
        ],
    )(acc, deg, xr, wl, wr, b)


def _fin_body(a_ref, d_ref, hr_ref, w_ref, b_ref, o_ref):
    deg = jnp.maximum(d_ref[0, :, :1] + d_ref[1, :, :1], 1.0)
    h = jnp.maximum((a_ref[0] + a_ref[1]) / deg + hr_ref[...], 0.0)
    o_ref[...] = jnp.dot(h, w_ref[...],
                         preferred_element_type=jnp.float32) + b_ref[...]


def _fin(acc, deg, hr, w, b):
    return pl.pallas_call(
        _fin_body,
        grid=(pl.cdiv(N, BR),),
        in_specs=[
            pl.BlockSpec((NC, BR, H), lambda i: (0, i, 0)),
            pl.BlockSpec((NC, BR, DEGW), lambda i: (0, i, 0)),
            pl.BlockSpec((BR, H), lambda i: (i, 0)),
            pl.BlockSpec((H, C), lambda i: (0, 0)),
            pl.BlockSpec((1, C), lambda i: (0, 0)),
        ],
        out_specs=pl.BlockSpec((BR, C), lambda i: (i, 0)),
        out_shape=jax.ShapeDtypeStruct((N, C), jnp.float32),
    )(acc, deg, hr, w, b)


NBUF = 2
GLEAD = 1
SLAG = NBUF - GLEAD
NOUTER = NCHUNK // NBUF
RPT_T = N // NS


def _pipeline(table, src_v, dst_v, rows_v, acc_s, gsem, ssem,
              ones_v=None, deg_s=None, dsem=None):
    with_deg = deg_s is not None

    def gstart(j, b):
        pltpu.async_copy(table.at[src_v.at[j]], rows_v.at[b], gsem.at[b])

    def step(j, b):
        if with_deg:
            pltpu.async_copy(ones_v, deg_s.at[dst_v.at[j]], dsem, add=True)
        pltpu.make_async_copy(
            table.at[src_v.at[j]], rows_v.at[b], gsem.at[b]).wait()
        pltpu.async_copy(
            rows_v.at[b], acc_s.at[dst_v.at[j]], ssem.at[b], add=True)

    def swait(j, b):
        pltpu.make_async_copy(
            rows_v.at[b], acc_s.at[dst_v.at[j]], ssem.at[b]).wait()

    def dwait(j):
        pltpu.make_async_copy(ones_v, deg_s.at[dst_v.at[j]], dsem).wait()

    for j in range(GLEAD):
        gstart(j, j % NBUF)
    for j in range(NBUF):
        if j >= SLAG:
            swait(j - SLAG, (j - SLAG) % NBUF)
        gstart(j + GLEAD, (j + GLEAD) % NBUF)
        step(j, j % NBUF)
        if with_deg and j >= 1:
            dwait(j - 1)

    def outer(i, carry):
        base = i * NBUF
        for b in range(NBUF):
            j = base + b
            swait(j - SLAG, (b - SLAG) % NBUF)
            gstart(j + GLEAD, (b + GLEAD) % NBUF)
            step(j, b)
            if with_deg:
                dwait(j - 1)
        return carry

    lax.fori_loop(1, NOUTER - 1, outer, 0)

    base = NCHUNK - NBUF
    for b in range(NBUF):
        j = base + b
        swait(j - SLAG, (j - SLAG) % NBUF)
        if j + GLEAD < NCHUNK:
            gstart(j + GLEAD, (j + GLEAD) % NBUF)
        step(j, b)
        if with_deg:
            dwait(j - 1)
    for j in range(NCHUNK - SLAG, NCHUNK):
        swait(j, j % NBUF)
    if with_deg:
        dwait(NCHUNK - 1)


def _agg_deg_body(table, ei_h, zrow_h, zdeg_h, acc_out, deg_out,
                  src_v, dst_v, rows_v, ones_v, table_s, acc_s, deg_s,
                  gsem, ssem, dsem):
    cid = lax.axis_index("c")
    sid = lax.axis_index("s")
    wid = sid * NC + cid
    pltpu.sync_copy(ei_h.at[0, wid], src_v)
    pltpu.sync_copy(ei_h.at[1, wid], dst_v)
    t0 = sid * RPT_T
    pltpu.sync_copy(table.at[pl.ds(t0, RPT_T)], table_s.at[pl.ds(t0, RPT_T)])
    r0 = sid * RPT
    pltpu.sync_copy(zrow_h, acc_s.at[pl.ds(r0, RPT)])
    pltpu.sync_copy(zdeg_h, deg_s.at[pl.ds(r0, RPT)])
    one = jnp.ones((DEGW,), jnp.float32)
    for r in range(CK):
        ones_v[r, :] = one
    plsc.subcore_barrier()
    _pipeline(table_s, src_v, dst_v, rows_v, acc_s, gsem, ssem,
              ones_v, deg_s, dsem)
    plsc.subcore_barrier()
    pltpu.sync_copy(acc_s.at[pl.ds(r0, RPT)], acc_out.at[cid, pl.ds(r0, RPT)])
    pltpu.sync_copy(deg_s.at[pl.ds(r0, RPT)], deg_out.at[cid, pl.ds(r0, RPT)])


def _agg_body(table, ei_h, zrow_h, acc_out,
              src_v, dst_v, rows_v, table_s, acc_s, gsem, ssem):
    cid = lax.axis_index("c")
    sid = lax.axis_index("s")
    wid = sid * NC + cid
    pltpu.sync_copy(ei_h.at[0, wid], src_v)
    pltpu.sync_copy(ei_h.at[1, wid], dst_v)
    t0 = sid * RPT_T
    pltpu.sync_copy(table.at[pl.ds(t0, RPT_T)], table_s.at[pl.ds(t0, RPT_T)])
    r0 = sid * RPT
    pltpu.sync_copy(zrow_h, acc_s.at[pl.ds(r0, RPT)])
    plsc.subcore_barrier()
    _pipeline(table_s, src_v, dst_v, rows_v, acc_s, gsem, ssem)
    plsc.subcore_barrier()
    pltpu.sync_copy(acc_s.at[pl.ds(r0, RPT)], acc_out.at[cid, pl.ds(r0, RPT)])


_SC_MESH = dict(core_axis_name="c", subcore_axis_name="s")


def _agg_deg(table, ei, zrow, zdeg):
    return pl.kernel(
        _agg_deg_body,
        out_type=(
            jax.ShapeDtypeStruct((NC, N_PAD, H), jnp.float32),
            jax.ShapeDtypeStruct((NC, N_PAD, DEGW), jnp.float32),
        ),
        mesh=plsc.VectorSubcoreMesh(**_SC_MESH),
        compiler_params=pltpu.CompilerParams(use_tc_tiling_on_sc=False),
        scratch_types=[
            pltpu.VMEM((NCHUNK, CK), jnp.int32),
            pltpu.VMEM((NCHUNK, CK), jnp.int32),
            pltpu.VMEM((NBUF, CK, H), jnp.float32),
            pltpu.VMEM((CK, DEGW), jnp.float32),
            pltpu.VMEM_SHARED((N, H), jnp.float32),
            pltpu.VMEM_SHARED((N_PAD, H), jnp.float32),
            pltpu.VMEM_SHARED((N_PAD, DEGW), jnp.float32),
            pltpu.SemaphoreType.DMA((NBUF,)),
            pltpu.SemaphoreType.DMA((NBUF,)),
            pltpu.SemaphoreType.DMA,
        ],
    )(table, ei, zrow, zdeg)


def _agg(table, ei, zrow):
    return pl.kernel(
        _agg_body,
        out_type=jax.ShapeDtypeStruct((NC, N_PAD, H), jnp.float32),
        mesh=plsc.VectorSubcoreMesh(**_SC_MESH),
        compiler_params=pltpu.CompilerParams(use_tc_tiling_on_sc=False),
        scratch_types=[
            pltpu.VMEM((NCHUNK, CK), jnp.int32),
            pltpu.VMEM((NCHUNK, CK), jnp.int32),
            pltpu.VMEM((NBUF, CK, H), jnp.float32),
            pltpu.VMEM_SHARED((N, H), jnp.float32),
            pltpu.VMEM_SHARED((N_PAD, H), jnp.float32),
            pltpu.SemaphoreType.DMA((NBUF,)),
            pltpu.SemaphoreType.DMA((NBUF,)),
        ],
    )(table, ei, zrow)


def kernel(x, edge_index, batch, W1l, b1, W1r, W2l, b2, W2r, Wlin, blin):
    ei = edge_index.reshape(2, NW, NCHUNK, CK)
    zrow = jnp.zeros((RPT, H), jnp.float32)
    zdeg = jnp.zeros((RPT, DEGW), jnp.float32)

    xl, xr1 = _lin1(x, W1l, W1r, b1.reshape(1, H))
    acc1, deg1 = _agg_deg(xl, ei, zrow, zdeg)
    hl, hr2 = _mid(acc1, deg1, xr1, W2l, W2r, b2.reshape(1, H))
    acc2 = _agg(hl, ei, zrow)
    return _fin(acc2, deg1, hr2, Wlin, blin.reshape(1, C))
